# baseline (device time: 388873 ns/iter reference)
import numpy as np
import jax
import jax.numpy as jnp
from jax import lax
from jax.experimental import pallas as pl
from jax.experimental.pallas import tpu as pltpu

N = 16
B_LOC = 2
SQ = 256
D = 768
H_LOC = 4
DH = 64
ROWS = B_LOC * SQ
HD_LOC = H_LOC * DH


def _rope_tables():
    inv = 1.0 / (10000.0 ** (np.arange(0, DH, 2) / DH))
    pos = np.arange(SQ)[:, None] * inv[None, :]
    cos = np.repeat(np.cos(pos), 2, axis=-1)
    sin = np.repeat(np.sin(pos), 2, axis=-1)
    cos = np.tile(cos, (B_LOC, H_LOC)).astype(np.float32)
    sin = np.tile(sin, (B_LOC, H_LOC)).astype(np.float32)
    return cos, sin


def _rot_cols(w):
    w3 = w.reshape(D, HD_LOC // 2, 2)
    return jnp.stack([-w3[..., 1], w3[..., 0]], axis=-1).reshape(D, HD_LOC)


def _body(x_ref, wq_ref, wk_ref, wv_ref, wqr_ref, wkr_ref, wo_ref,
          cos_ref, sin_ref, out_ref,
          xg, part, rsb, ag_send, ag_recv, rs_send, rs_recv):
    my = lax.axis_index("i")
    left = lax.rem(my + N - 1, N)
    right = lax.rem(my + 1, N)

    barrier = pltpu.get_barrier_semaphore()
    pl.semaphore_signal(barrier, inc=1, device_id=(left,),
                        device_id_type=pl.DeviceIdType.MESH)
    pl.semaphore_signal(barrier, inc=1, device_id=(right,),
                        device_id_type=pl.DeviceIdType.MESH)
    pl.semaphore_wait(barrier, 2)

    xg[my] = x_ref[...]

    for s in range(N - 1):
        slot = lax.rem(my - s + 2 * N, N)
        rdma = pltpu.make_async_remote_copy(
            src_ref=xg.at[slot],
            dst_ref=xg.at[slot],
            send_sem=ag_send.at[s],
            recv_sem=ag_recv.at[s],
            device_id=(right,),
            device_id_type=pl.DeviceIdType.MESH,
        )
        rdma.start()
        rdma.wait()

    cos = cos_ref[...]
    sin = sin_ref[...]
    wq = wq_ref[...]
    wk = wk_ref[...]
    wv = wv_ref[...]
    wqr = wqr_ref[...]
    wkr = wkr_ref[...]
    wo = wo_ref[...]

    def compute_chunk(d, carry):
        xd = xg[d]
        q = jnp.dot(xd, wq, preferred_element_type=jnp.float32)
        qr = jnp.dot(xd, wqr, preferred_element_type=jnp.float32)
        k = jnp.dot(xd, wk, preferred_element_type=jnp.float32)
        kr = jnp.dot(xd, wkr, preferred_element_type=jnp.float32)
        v = jnp.dot(xd, wv,
                    preferred_element_type=jnp.float32).astype(jnp.bfloat16)
        qq = (q * cos + qr * sin).astype(jnp.bfloat16)
        kk = (k * cos + kr * sin).astype(jnp.bfloat16)
        bats = []
        for b in range(B_LOC):
            rows = slice(b * SQ, (b + 1) * SQ)
            heads = []
            for h in range(H_LOC):
                cols = slice(h * DH, (h + 1) * DH)
                qb = qq[rows, cols]
                kb = kk[rows, cols]
                vb = v[rows, cols]
                sc = lax.dot_general(
                    qb, kb, (((1,), (1,)), ((), ())),
                    preferred_element_type=jnp.float32) * 0.125
                m = jnp.max(sc, axis=-1, keepdims=True)
                e = jnp.exp(sc - m)
                w = (e / jnp.sum(e, axis=-1, keepdims=True)).astype(jnp.bfloat16)
                heads.append(jnp.dot(w, vb,
                                     preferred_element_type=jnp.float32))
            bats.append(jnp.concatenate(heads, axis=1))
        ctx = jnp.concatenate(bats, axis=0).astype(jnp.bfloat16)
        part[d] = jnp.dot(
            ctx, wo, preferred_element_type=jnp.float32).astype(jnp.bfloat16)
        return carry

    lax.fori_loop(0, N, compute_chunk, 0)

    for s in range(N - 1):
        src_slot = lax.rem(my - 1 - s + 2 * N, N)
        rdma = pltpu.make_async_remote_copy(
            src_ref=part.at[src_slot],
            dst_ref=rsb.at[s],
            send_sem=rs_send.at[s],
            recv_sem=rs_recv.at[s],
            device_id=(right,),
            device_id_type=pl.DeviceIdType.MESH,
        )
        rdma.start()
        rdma.wait()
        acc_slot = lax.rem(my - 2 - s + 2 * N, N)
        part[acc_slot] = part[acc_slot] + rsb[s]

    out_ref[...] = part[my]


def kernel(x, Wq, Wk, Wv, Wo):
    bf = jnp.bfloat16
    x2 = x.astype(bf).reshape(ROWS, D)
    wq = Wq.astype(bf)
    wk = Wk.astype(bf)
    wv = Wv.astype(bf)
    wo = Wo.astype(bf)
    wqr = _rot_cols(wq)
    wkr = _rot_cols(wk)
    cos_t, sin_t = _rope_tables()
    cos_t = jnp.asarray(cos_t)
    sin_t = jnp.asarray(sin_t)

    vmem = pl.BlockSpec(memory_space=pltpu.VMEM)
    out = pl.pallas_call(
        _body,
        out_shape=jax.ShapeDtypeStruct((ROWS, D), bf),
        in_specs=[vmem] * 9,
        out_specs=vmem,
        scratch_shapes=[
            pltpu.VMEM((N, ROWS, D), bf),
            pltpu.VMEM((N, ROWS, D), bf),
            pltpu.VMEM((N - 1, ROWS, D), bf),
            pltpu.SemaphoreType.DMA((N - 1,)),
            pltpu.SemaphoreType.DMA((N - 1,)),
            pltpu.SemaphoreType.DMA((N - 1,)),
            pltpu.SemaphoreType.DMA((N - 1,)),
        ],
        compiler_params=pltpu.CompilerParams(
            collective_id=0,
            vmem_limit_bytes=100 * 1024 * 1024,
        ),
    )(x2, wq, wk, wv, wqr, wkr, wo, cos_t, sin_t)
    return out.reshape(B_LOC, SQ, D)


# device time: 193150 ns/iter; 2.0133x vs baseline; 2.0133x over previous
import numpy as np
import jax
import jax.numpy as jnp
from jax import lax
from jax.experimental import pallas as pl
from jax.experimental.pallas import tpu as pltpu

N = 16
B_LOC = 2
SQ = 256
D = 768
H_LOC = 4
DH = 64
ROWS = B_LOC * SQ
HD_LOC = H_LOC * DH
R_STEPS = N // 2
L_STEPS = N // 2 - 1


def _rope_tables():
    inv = 1.0 / (10000.0 ** (np.arange(0, DH, 2) / DH))
    pos = np.arange(SQ)[:, None] * inv[None, :]
    cos = np.repeat(np.cos(pos), 2, axis=-1)
    sin = np.repeat(np.sin(pos), 2, axis=-1)
    cos = np.tile(cos, (B_LOC, H_LOC)).astype(np.float32)
    sin = np.tile(sin, (B_LOC, H_LOC)).astype(np.float32)
    return cos, sin


def _rot_cols(w):
    w3 = w.reshape(D, HD_LOC // 2, 2)
    return jnp.stack([-w3[..., 1], w3[..., 0]], axis=-1).reshape(D, HD_LOC)


def _body(x_ref, wqkv_ref, wo_ref, cos_ref, sin_ref, out_ref,
          xg, part, rsbR, rsbL,
          agR_send, agR_recv, agL_send, agL_recv,
          rsR_send, rsR_recv, rsL_send, rsL_recv):
    my = lax.axis_index("i")
    left = lax.rem(my + N - 1, N)
    right = lax.rem(my + 1, N)

    barrier = pltpu.get_barrier_semaphore()
    pl.semaphore_signal(barrier, inc=1, device_id=(left,),
                        device_id_type=pl.DeviceIdType.MESH)
    pl.semaphore_signal(barrier, inc=1, device_id=(right,),
                        device_id_type=pl.DeviceIdType.MESH)
    pl.semaphore_wait(barrier, 2)

    cos = cos_ref[...]
    sin = sin_ref[...]
    wqkv = wqkv_ref[...]
    wo = wo_ref[...]

    def compute_chunk(d):
        xd = xg[d]
        proj = jnp.dot(xd, wqkv, preferred_element_type=jnp.float32)
        q = proj[:, 0 * HD_LOC:1 * HD_LOC]
        qr = proj[:, 1 * HD_LOC:2 * HD_LOC]
        k = proj[:, 2 * HD_LOC:3 * HD_LOC]
        kr = proj[:, 3 * HD_LOC:4 * HD_LOC]
        v = proj[:, 4 * HD_LOC:5 * HD_LOC].astype(jnp.bfloat16)
        qq = (q * cos + qr * sin).astype(jnp.bfloat16)
        kk = (k * cos + kr * sin).astype(jnp.bfloat16)
        bats = []
        for b in range(B_LOC):
            rows = slice(b * SQ, (b + 1) * SQ)
            heads = []
            for h in range(H_LOC):
                cols = slice(h * DH, (h + 1) * DH)
                qb = qq[rows, cols]
                kb = kk[rows, cols]
                vb = v[rows, cols]
                sc = lax.dot_general(
                    qb, kb, (((1,), (1,)), ((), ())),
                    preferred_element_type=jnp.float32) * 0.125
                m = jnp.max(sc, axis=-1, keepdims=True)
                e = jnp.exp(sc - m)
                w = (e / jnp.sum(e, axis=-1, keepdims=True)).astype(jnp.bfloat16)
                heads.append(jnp.dot(w, vb,
                                     preferred_element_type=jnp.float32))
            bats.append(jnp.concatenate(heads, axis=1))
        ctx = jnp.concatenate(bats, axis=0).astype(jnp.bfloat16)
        part[d] = jnp.dot(
            ctx, wo, preferred_element_type=jnp.float32).astype(jnp.bfloat16)

    def ag_right(s):
        slot = lax.rem(my - s + 2 * N, N)
        return pltpu.make_async_remote_copy(
            src_ref=xg.at[slot], dst_ref=xg.at[slot],
            send_sem=agR_send.at[s], recv_sem=agR_recv.at[s],
            device_id=(right,), device_id_type=pl.DeviceIdType.MESH)

    def ag_left(s):
        slot = lax.rem(my + s, N)
        return pltpu.make_async_remote_copy(
            src_ref=xg.at[slot], dst_ref=xg.at[slot],
            send_sem=agL_send.at[s], recv_sem=agL_recv.at[s],
            device_id=(left,), device_id_type=pl.DeviceIdType.MESH)

    def rs_right(t):
        slot = lax.rem(my + R_STEPS - t + 2 * N, N)
        return pltpu.make_async_remote_copy(
            src_ref=part.at[slot], dst_ref=rsbR.at[t],
            send_sem=rsR_send.at[t], recv_sem=rsR_recv.at[t],
            device_id=(right,), device_id_type=pl.DeviceIdType.MESH)

    def rs_left(t):
        slot = lax.rem(my - L_STEPS + t + 2 * N, N)
        return pltpu.make_async_remote_copy(
            src_ref=part.at[slot], dst_ref=rsbL.at[t],
            send_sem=rsL_send.at[t], recv_sem=rsL_recv.at[t],
            device_id=(left,), device_id_type=pl.DeviceIdType.MESH)

    xg[my] = x_ref[...]
    ag_right(0).start()
    ag_left(0).start()
    compute_chunk(my)

    for s in range(R_STEPS):
        ag_right(s).wait_recv()
        if s + 1 < R_STEPS:
            ag_right(s + 1).start()
        if s < L_STEPS:
            ag_left(s).wait_recv()
            if s + 1 < L_STEPS:
                ag_left(s + 1).start()
        compute_chunk(lax.rem(my - 1 - s + 2 * N, N))
        if s < L_STEPS:
            compute_chunk(lax.rem(my + 1 + s, N))

    rs_right(0).start()
    rs_left(0).start()
    for t in range(1, R_STEPS):
        rs_right(t - 1).wait_recv()
        slotR = lax.rem(my + R_STEPS - t + 2 * N, N)
        part[slotR] = part[slotR] + rsbR[t - 1]
        rs_right(t).start()
        if t < L_STEPS:
            rs_left(t - 1).wait_recv()
            slotL = lax.rem(my - L_STEPS + t + 2 * N, N)
            part[slotL] = part[slotL] + rsbL[t - 1]
            rs_left(t).start()

    rs_right(R_STEPS - 1).wait_recv()
    rs_left(L_STEPS - 1).wait_recv()
    out_ref[...] = part[my] + rsbR[R_STEPS - 1] + rsbL[L_STEPS - 1]

    for s in range(R_STEPS):
        ag_right(s).wait_send()
        rs_right(s).wait_send()
    for s in range(L_STEPS):
        ag_left(s).wait_send()
        rs_left(s).wait_send()


def kernel(x, Wq, Wk, Wv, Wo):
    bf = jnp.bfloat16
    x2 = x.astype(bf).reshape(ROWS, D)
    wq = Wq.astype(bf)
    wk = Wk.astype(bf)
    wv = Wv.astype(bf)
    wo = Wo.astype(bf)
    wqkv = jnp.concatenate(
        [wq, _rot_cols(wq), wk, _rot_cols(wk), wv], axis=1)
    cos_t, sin_t = _rope_tables()
    cos_t = jnp.asarray(cos_t)
    sin_t = jnp.asarray(sin_t)

    vmem = pl.BlockSpec(memory_space=pltpu.VMEM)
    out = pl.pallas_call(
        _body,
        out_shape=jax.ShapeDtypeStruct((ROWS, D), bf),
        in_specs=[vmem] * 5,
        out_specs=vmem,
        scratch_shapes=[
            pltpu.VMEM((N, ROWS, D), bf),
            pltpu.VMEM((N, ROWS, D), bf),
            pltpu.VMEM((R_STEPS, ROWS, D), bf),
            pltpu.VMEM((L_STEPS, ROWS, D), bf),
            pltpu.SemaphoreType.DMA((R_STEPS,)),
            pltpu.SemaphoreType.DMA((R_STEPS,)),
            pltpu.SemaphoreType.DMA((L_STEPS,)),
            pltpu.SemaphoreType.DMA((L_STEPS,)),
            pltpu.SemaphoreType.DMA((R_STEPS,)),
            pltpu.SemaphoreType.DMA((R_STEPS,)),
            pltpu.SemaphoreType.DMA((L_STEPS,)),
            pltpu.SemaphoreType.DMA((L_STEPS,)),
        ],
        compiler_params=pltpu.CompilerParams(
            collective_id=0,
            vmem_limit_bytes=110 * 1024 * 1024,
        ),
    )(x2, wqkv, wo, cos_t, sin_t)
    return out.reshape(B_LOC, SQ, D)


# device time: 168238 ns/iter; 2.3114x vs baseline; 1.1481x over previous
import numpy as np
import jax
import jax.numpy as jnp
from jax import lax
from jax.experimental import pallas as pl
from jax.experimental.pallas import tpu as pltpu

N = 16
B_LOC = 2
SQ = 256
D = 768
H_LOC = 4
DH = 64
ROWS = B_LOC * SQ
HD_LOC = H_LOC * DH
BLOB = D + HD_LOC
R_STEPS = N // 2
L_STEPS = N // 2 - 1


def _rope_tables():
    inv = 1.0 / (10000.0 ** (np.arange(0, DH, 2) / DH))
    pos = np.arange(SQ)[:, None] * inv[None, :]
    cos = np.repeat(np.cos(pos), 2, axis=-1)
    sin = np.repeat(np.sin(pos), 2, axis=-1)
    cos = np.tile(cos, (B_LOC, H_LOC)).astype(np.float32)
    sin = np.tile(sin, (B_LOC, H_LOC)).astype(np.float32)
    return cos, sin


def _rot_matrix():
    r = np.zeros((HD_LOC, HD_LOC), np.float32)
    for k in range(HD_LOC // 2):
        r[2 * k + 1, 2 * k] = -1.0
        r[2 * k, 2 * k + 1] = 1.0
    return r


def _body(x_ref, blob_ref, cos_ref, sin_ref, rot_ref, out_ref,
          wg, acc, agR_send, agR_recv, agL_send, agL_recv):
    my = lax.axis_index("i")
    left = lax.rem(my + N - 1, N)
    right = lax.rem(my + 1, N)

    barrier = pltpu.get_barrier_semaphore()
    pl.semaphore_signal(barrier, inc=1, device_id=(left,),
                        device_id_type=pl.DeviceIdType.MESH)
    pl.semaphore_signal(barrier, inc=1, device_id=(right,),
                        device_id_type=pl.DeviceIdType.MESH)
    pl.semaphore_wait(barrier, 2)

    x = x_ref[...]
    cos = cos_ref[...]
    sin = sin_ref[...]
    rot = rot_ref[...]

    def compute_chunk(d, first):
        blob = wg[d]
        wqkv = blob[:D, :]
        wo_d = blob[D:, :]
        proj = jnp.dot(x, wqkv, preferred_element_type=jnp.float32)
        q = proj[:, 0 * HD_LOC:1 * HD_LOC]
        k = proj[:, 1 * HD_LOC:2 * HD_LOC]
        v = proj[:, 2 * HD_LOC:3 * HD_LOC].astype(jnp.bfloat16)
        qr = jnp.dot(q.astype(jnp.bfloat16), rot,
                     preferred_element_type=jnp.float32)
        kr = jnp.dot(k.astype(jnp.bfloat16), rot,
                     preferred_element_type=jnp.float32)
        qq = (q * cos + qr * sin).astype(jnp.bfloat16)
        kk = (k * cos + kr * sin).astype(jnp.bfloat16)
        bats = []
        for b in range(B_LOC):
            rows = slice(b * SQ, (b + 1) * SQ)
            heads = []
            for h in range(H_LOC):
                cols = slice(h * DH, (h + 1) * DH)
                qb = qq[rows, cols]
                kb = kk[rows, cols]
                vb = v[rows, cols]
                sc = lax.dot_general(
                    qb, kb, (((1,), (1,)), ((), ())),
                    preferred_element_type=jnp.float32) * 0.125
                m = jnp.max(sc, axis=-1, keepdims=True)
                e = jnp.exp(sc - m)
                w = (e / jnp.sum(e, axis=-1, keepdims=True)).astype(jnp.bfloat16)
                heads.append(jnp.dot(w, vb,
                                     preferred_element_type=jnp.float32))
            bats.append(jnp.concatenate(heads, axis=1))
        ctx = jnp.concatenate(bats, axis=0).astype(jnp.bfloat16)
        contrib = jnp.dot(ctx, wo_d, preferred_element_type=jnp.float32)
        if first:
            acc[...] = contrib
        else:
            acc[...] = acc[...] + contrib

    def ag_right(s):
        slot = lax.rem(my - s + 2 * N, N)
        return pltpu.make_async_remote_copy(
            src_ref=wg.at[slot], dst_ref=wg.at[slot],
            send_sem=agR_send.at[s], recv_sem=agR_recv.at[s],
            device_id=(right,), device_id_type=pl.DeviceIdType.MESH)

    def ag_left(s):
        slot = lax.rem(my + s, N)
        return pltpu.make_async_remote_copy(
            src_ref=wg.at[slot], dst_ref=wg.at[slot],
            send_sem=agL_send.at[s], recv_sem=agL_recv.at[s],
            device_id=(left,), device_id_type=pl.DeviceIdType.MESH)

    wg[my] = blob_ref[...]
    ag_right(0).start()
    ag_left(0).start()
    compute_chunk(my, first=True)

    for s in range(R_STEPS):
        ag_right(s).wait_recv()
        if s + 1 < R_STEPS:
            ag_right(s + 1).start()
        if s < L_STEPS:
            ag_left(s).wait_recv()
            if s + 1 < L_STEPS:
                ag_left(s + 1).start()
        compute_chunk(lax.rem(my - 1 - s + 2 * N, N), first=False)
        if s < L_STEPS:
            compute_chunk(lax.rem(my + 1 + s, N), first=False)

    out_ref[...] = acc[...]

    for s in range(R_STEPS):
        ag_right(s).wait_send()
    for s in range(L_STEPS):
        ag_left(s).wait_send()


def kernel(x, Wq, Wk, Wv, Wo):
    bf = jnp.bfloat16
    x2 = x.astype(bf).reshape(ROWS, D)
    blob = jnp.concatenate(
        [jnp.concatenate([Wq.astype(bf), Wk.astype(bf), Wv.astype(bf)],
                         axis=1),
         Wo.astype(bf)], axis=0)
    cos_t, sin_t = _rope_tables()
    cos_t = jnp.asarray(cos_t)
    sin_t = jnp.asarray(sin_t)
    rot = jnp.asarray(_rot_matrix()).astype(bf)

    vmem = pl.BlockSpec(memory_space=pltpu.VMEM)
    out = pl.pallas_call(
        _body,
        out_shape=jax.ShapeDtypeStruct((ROWS, D), jnp.float32),
        in_specs=[vmem] * 5,
        out_specs=vmem,
        scratch_shapes=[
            pltpu.VMEM((N, BLOB, D), bf),
            pltpu.VMEM((ROWS, D), jnp.float32),
            pltpu.SemaphoreType.DMA((R_STEPS,)),
            pltpu.SemaphoreType.DMA((R_STEPS,)),
            pltpu.SemaphoreType.DMA((L_STEPS,)),
            pltpu.SemaphoreType.DMA((L_STEPS,)),
        ],
        compiler_params=pltpu.CompilerParams(
            collective_id=0,
            vmem_limit_bytes=110 * 1024 * 1024,
        ),
    )(x2, blob, cos_t, sin_t, rot)
    return out.reshape(B_LOC, SQ, D)
